# Initial kernel scaffold; baseline (speedup 1.0000x reference)
import jax
import jax.numpy as jnp
from jax.experimental import pallas as pl


def kernel(x, edge_index, edge_weight, batch, W0, b0, p0, W1c, b1c, p1, W2c, b2c, p2, M1W, M1b, M2W, M2b, M3W, M3b):
    raise NotImplementedError("write your pallas kernel here")


# speedup vs baseline: 12.0772x; 12.0772x over previous
import functools

import jax
import jax.numpy as jnp
from jax import lax
from jax.experimental import pallas as pl
from jax.experimental.pallas import tpu as pltpu
from jax.experimental.pallas import tpu_sc as plsc

N = 10000
NP = 10240
E = 320000
EPT = 10240
EP_TOTAL = EPT * 32
C = 128
B = 64
INT_MIN = -2147483648

_mesh = plsc.VectorSubcoreMesh(core_axis_name="c", subcore_axis_name="s")
_sc_params = pltpu.CompilerParams(needs_layout_passes=False)


@functools.partial(
    pl.kernel,
    out_type=(jax.ShapeDtypeStruct((EP_TOTAL,), jnp.float32),
              jax.ShapeDtypeStruct((32, NP), jnp.float32)),
    mesh=_mesh,
    scratch_types=[pltpu.VMEM((EPT,), jnp.int32),
                   pltpu.VMEM((EPT,), jnp.int32),
                   pltpu.VMEM((EPT,), jnp.float32),
                   pltpu.VMEM((NP,), jnp.float32),
                   pltpu.VMEM((NP,), jnp.float32),
                   pltpu.VMEM((EPT,), jnp.float32)],
    compiler_params=_sc_params,
)
def _sc_deg(src_h, dst_h, w_h, alive_h, weff_h, degp_h,
            src_v, dst_v, w_v, alive_v, deg_v, weff_v):
    wid = lax.axis_index("c") * 16 + lax.axis_index("s")
    base = wid * EPT
    pltpu.sync_copy(src_h.at[pl.ds(base, EPT)], src_v)
    pltpu.sync_copy(dst_h.at[pl.ds(base, EPT)], dst_v)
    pltpu.sync_copy(w_h.at[pl.ds(base, EPT)], w_v)
    pltpu.sync_copy(alive_h, alive_v)

    zeros16 = jnp.zeros((16,), jnp.float32)

    def zbody(i, carry):
        deg_v[pl.ds(i * 16, 16)] = zeros16
        return carry
    lax.fori_loop(0, NP // 16, zbody, 0)

    def ebody(i, carry):
        sl = pl.ds(i * 16, 16)
        s = src_v[sl]
        d = dst_v[sl]
        w = w_v[sl]
        a_s = plsc.load_gather(alive_v, [s])
        a_d = plsc.load_gather(alive_v, [d])
        we = w * a_s * a_d
        weff_v[sl] = we
        plsc.addupdate_scatter(deg_v, [d], we)
        return carry
    lax.fori_loop(0, EPT // 16, ebody, 0)

    pltpu.sync_copy(weff_v, weff_h.at[pl.ds(base, EPT)])
    pltpu.sync_copy(deg_v, degp_h.at[wid])


@functools.partial(
    pl.kernel,
    out_type=jax.ShapeDtypeStruct((2, NP, C), jnp.float32),
    mesh=_mesh,
    scratch_types=[pltpu.VMEM((EPT,), jnp.int32),
                   pltpu.VMEM((EPT,), jnp.int32),
                   pltpu.VMEM((EPT,), jnp.float32),
                   pltpu.VMEM((128, C), jnp.float32),
                   pltpu.VMEM((128,), jnp.int32),
                   pltpu.VMEM_SHARED((NP, C), jnp.float32),
                   pltpu.SemaphoreType.DMA],
    compiler_params=_sc_params,
)
def _sc_spmm(g_h, src_h, dst_h, w_h, zero_h, acc_out,
             src_v, dst_v, w_v, rows, dst_stage, acc_sh, sem):
    cid = lax.axis_index("c")
    sid = lax.axis_index("s")
    wid = cid * 16 + sid
    base = wid * EPT
    rstripe = pl.ds(sid * (NP // 16), NP // 16)
    pltpu.sync_copy(zero_h.at[rstripe], acc_sh.at[rstripe])
    pltpu.sync_copy(src_h.at[pl.ds(base, EPT)], src_v)
    pltpu.sync_copy(dst_h.at[pl.ds(base, EPT)], dst_v)
    pltpu.sync_copy(w_h.at[pl.ds(base, EPT)], w_v)
    plsc.subcore_barrier()

    def jbody(j, carry):
        eb = j * 128
        pltpu.async_copy(g_h.at[src_v.at[pl.ds(eb, 128)]], rows, sem).wait()
        for b2 in range(8):
            dst_stage[pl.ds(b2 * 16, 16)] = dst_v[pl.ds(eb + b2 * 16, 16)]

        def ebody(e, ecarry):
            ws = plsc.load_gather(w_v, [jnp.full((16,), eb + e, jnp.int32)])
            for b3 in range(8):
                cs = pl.ds(b3 * 16, 16)
                rows[e, cs] = rows[e, cs] * ws
            return ecarry
        lax.fori_loop(0, 128, ebody, 0)
        pltpu.sync_copy(rows, acc_sh.at[dst_stage], add=True)
        return carry
    lax.fori_loop(0, EPT // 128, jbody, 0)

    plsc.subcore_barrier()
    pltpu.sync_copy(acc_sh.at[rstripe], acc_out.at[cid, rstripe])


BR = 1280
GR = NP // BR
RB2 = 256
GR2 = NP // RB2
_NEG = -3.0e38


def _rows(width):
    return pl.BlockSpec((BR, width), lambda i: (i, 0))


def _full2(a, b):
    return pl.BlockSpec((a, b), lambda i: (0, 0))


def _prep_body(matmul, refs):
    if matmul:
        h_ref, w_ref, b_ref, degp_ref, alive_ref, g_ref, self_ref, dinv_ref = refs
        h = jnp.dot(h_ref[...], w_ref[...],
                    preferred_element_type=jnp.float32) + b_ref[...]
    else:
        h_ref, degp_ref, alive_ref, g_ref, self_ref, dinv_ref = refs
        h = h_ref[...]
    alive = alive_ref[...]
    deg = jnp.sum(degp_ref[...], axis=0)[:, None] + alive
    dinv = lax.rsqrt(jnp.maximum(deg, 1e-12))
    g_ref[...] = h * dinv
    self_ref[...] = h * (alive * dinv * dinv)
    dinv_ref[...] = dinv


_prep_out = [jax.ShapeDtypeStruct((NP, C), jnp.float32),
             jax.ShapeDtypeStruct((NP, C), jnp.float32),
             jax.ShapeDtypeStruct((NP, 1), jnp.float32)]
_prep_out_specs = [_rows(C), _rows(C), _rows(1)]

_tc_prep_mm = pl.pallas_call(
    lambda *refs: _prep_body(True, refs),
    grid=(GR,),
    in_specs=[_rows(C), _full2(C, C), _full2(1, C),
              pl.BlockSpec((32, BR), lambda i: (0, i)), _rows(1)],
    out_specs=_prep_out_specs,
    out_shape=_prep_out)

_tc_prep = pl.pallas_call(
    lambda *refs: _prep_body(False, refs),
    grid=(GR,),
    in_specs=[_rows(C), pl.BlockSpec((32, BR), lambda i: (0, i)), _rows(1)],
    out_specs=_prep_out_specs,
    out_shape=_prep_out)


def _outscore_body(acc_ref, dinv_ref, self_ref, p_ref, out_ref, score_ref):
    accp = acc_ref[...]
    out = jnp.maximum(dinv_ref[...] * (accp[0] + accp[1]) + self_ref[...], 0.0)
    out_ref[...] = out
    p = p_ref[...]
    pn = jnp.sqrt(jnp.sum(p * p))
    score_ref[...] = jnp.dot(out, p, preferred_element_type=jnp.float32) / (pn + 1e-12)


_tc_outscore = pl.pallas_call(
    _outscore_body,
    grid=(GR,),
    in_specs=[pl.BlockSpec((2, BR, C), lambda i: (0, i, 0)),
              _rows(1), _rows(C), _full2(C, 1)],
    out_specs=[_rows(C), _rows(1)],
    out_shape=[jax.ShapeDtypeStruct((NP, C), jnp.float32),
               jax.ShapeDtypeStruct((NP, 1), jnp.float32)])


def _sel_body(k, score_ref, alive_ref, sel_ref):
    score = score_ref[...]
    bits = lax.bitcast_convert_type(score, jnp.int32)
    key = jnp.where(bits >= 0, bits, bits ^ jnp.int32(0x7FFFFFFF))
    key = jnp.where(alive_ref[...] > 0, key, jnp.int32(INT_MIN))

    def bis(_, lohi):
        lo, hi = lohi
        mid = (lo & hi) + ((lo ^ hi) >> 1)
        cnt = jnp.sum((key >= mid).astype(jnp.int32))
        big = cnt >= k
        return jnp.where(big, mid, lo), jnp.where(big, hi, mid)
    lo, _ = lax.fori_loop(0, 32, bis,
                          (jnp.int32(INT_MIN), jnp.int32(2147483647)))
    tstar = lo
    c1 = jnp.sum((key > tstar).astype(jnp.int32))
    m = k - c1
    ties = key == tstar
    idx = lax.broadcasted_iota(jnp.int32, key.shape, 0)

    def bis2(_, lohi):
        lo2, hi2 = lohi
        mid = (lo2 + hi2) // 2
        cnt = jnp.sum((ties & (idx < mid)).astype(jnp.int32))
        ge = cnt >= m
        return jnp.where(ge, lo2, mid), jnp.where(ge, mid, hi2)
    _, u = lax.fori_loop(0, 15, bis2, (jnp.int32(0), jnp.int32(NP)))
    sel = (key > tstar) | (ties & (idx < u) & (m > 0))
    sel_ref[...] = sel.astype(jnp.float32)


def _mk_sel(k):
    return pl.pallas_call(
        functools.partial(_sel_body, k),
        out_shape=jax.ShapeDtypeStruct((NP, 1), jnp.float32))


_tc_sel0 = _mk_sel(5000)
_tc_sel1 = _mk_sel(2500)
_tc_sel2 = _mk_sel(1250)


def _gather_readout(has_next, refs):
    if has_next:
        (out_ref, score_ref, sel_ref, batch_ref, wn_ref, bn_ref,
         hnext_ref, hn_ref, msum_ref, cnt_ref, smax_ref, r_ref) = refs
    else:
        (out_ref, score_ref, sel_ref, batch_ref,
         hnext_ref, msum_ref, cnt_ref, smax_ref, r_ref) = refs
    i = pl.program_id(0)
    sel = sel_ref[...]
    hnext = out_ref[...] * (jnp.tanh(score_ref[...]) * sel)
    hnext_ref[...] = hnext
    if has_next:
        hn_ref[...] = jnp.dot(hnext, wn_ref[...],
                              preferred_element_type=jnp.float32) + bn_ref[...]

    @pl.when(i == 0)
    def _():
        msum_ref[...] = jnp.zeros_like(msum_ref)
        cnt_ref[...] = jnp.zeros_like(cnt_ref)
        smax_ref[...] = jnp.full_like(smax_ref, _NEG)

    bat = batch_ref[...]
    oh = ((jnp.reshape(bat, (1, RB2))
           == lax.broadcasted_iota(jnp.int32, (B, RB2), 0)).astype(jnp.float32)
          * jnp.reshape(sel, (1, RB2)))
    msum_ref[...] += jnp.dot(oh, hnext, preferred_element_type=jnp.float32)
    cnt_ref[...] += jnp.sum(oh, axis=1)[:, None]

    hm = jnp.where(sel > 0, hnext, _NEG)

    def mb(bb, carry):
        mrow = jnp.where(bat == bb, 0.0, _NEG)
        val = jnp.max(hm + mrow, axis=0)[None, :]
        cur = smax_ref[pl.ds(bb, 1), :]
        smax_ref[pl.ds(bb, 1), :] = jnp.maximum(cur, val)
        return carry
    lax.fori_loop(0, B, mb, 0, unroll=False)

    @pl.when(i == GR2 - 1)
    def _():
        cnt = cnt_ref[...]
        mean = msum_ref[...] / jnp.maximum(cnt, 1.0)
        smax = jnp.where(cnt > 0, smax_ref[...], 0.0)
        r_ref[...] = jnp.concatenate([mean, smax], axis=1)


def _rows2(width):
    return pl.BlockSpec((RB2, width), lambda i: (i, 0))


_gr_common_outs = [jax.ShapeDtypeStruct((B, C), jnp.float32),
                   jax.ShapeDtypeStruct((B, 1), jnp.float32),
                   jax.ShapeDtypeStruct((B, C), jnp.float32),
                   jax.ShapeDtypeStruct((B, 2 * C), jnp.float32)]
_gr_common_out_specs = [_full2(B, C), _full2(B, 1), _full2(B, C),
                        _full2(B, 2 * C)]

_tc_next = pl.pallas_call(
    lambda *refs: _gather_readout(True, refs),
    grid=(GR2,),
    in_specs=[_rows2(C), _rows2(1), _rows2(1), _rows2(1),
              _full2(C, C), _full2(1, C)],
    out_specs=[_rows2(C), _rows2(C)] + _gr_common_out_specs,
    out_shape=[jax.ShapeDtypeStruct((NP, C), jnp.float32),
               jax.ShapeDtypeStruct((NP, C), jnp.float32)] + _gr_common_outs)

_tc_last = pl.pallas_call(
    lambda *refs: _gather_readout(False, refs),
    grid=(GR2,),
    in_specs=[_rows2(C), _rows2(1), _rows2(1), _rows2(1)],
    out_specs=[_rows2(C)] + _gr_common_out_specs,
    out_shape=[jax.ShapeDtypeStruct((NP, C), jnp.float32)] + _gr_common_outs)


def _mlp_body(r0_ref, r1_ref, r2_ref, m1w_ref, m1b_ref, m2w_ref, m2b_ref,
              m3w_ref, m3b_ref, z_ref):
    r = r0_ref[...] + r1_ref[...] + r2_ref[...]
    z = jnp.maximum(jnp.dot(r, m1w_ref[...],
                            preferred_element_type=jnp.float32) + m1b_ref[...], 0.0)
    z = jnp.maximum(jnp.dot(z, m2w_ref[...],
                            preferred_element_type=jnp.float32) + m2b_ref[...], 0.0)
    z_ref[...] = jnp.dot(z, m3w_ref[...],
                         preferred_element_type=jnp.float32) + m3b_ref[...]


_tc_mlp = pl.pallas_call(
    _mlp_body, out_shape=jax.ShapeDtypeStruct((B, 1), jnp.float32))


def kernel(x, edge_index, edge_weight, batch, W0, b0, p0, W1c, b1c, p1,
           W2c, b2c, p2, M1W, M1b, M2W, M2b, M3W, M3b):
    f32 = jnp.float32
    src = edge_index[0]
    dst = edge_index[1]
    epad = EP_TOTAL - E
    src_p = jnp.concatenate([src, jnp.zeros((epad,), jnp.int32)])
    dst_p = jnp.concatenate([dst, jnp.zeros((epad,), jnp.int32)])
    w_p = jnp.concatenate([edge_weight.astype(f32), jnp.zeros((epad,), f32)])
    x_p = jnp.concatenate([x.astype(f32), jnp.zeros((NP - N, C), f32)], axis=0)
    batch_p = jnp.concatenate(
        [batch, jnp.full((NP - N,), B - 1, jnp.int32)])[:, None]
    alive0 = jnp.concatenate([jnp.ones((N, 1), f32), jnp.zeros((NP - N, 1), f32)])
    zero_nc = jnp.zeros((NP, C), f32)

    weff0, degp0 = _sc_deg(src_p, dst_p, w_p, alive0[:, 0])
    g0, self0, dinv0 = _tc_prep_mm(x_p, W0, b0[None, :], degp0, alive0)
    acc0 = _sc_spmm(g0, src_p, dst_p, weff0, zero_nc)
    out0, score0 = _tc_outscore(acc0, dinv0, self0, p0[:, None])
    alive1 = _tc_sel0(score0, alive0)
    _, h1, _, _, _, r1 = _tc_next(out0, score0, alive1, batch_p, W1c,
                                  b1c[None, :])

    weff1, degp1 = _sc_deg(src_p, dst_p, weff0, alive1[:, 0])
    g1, self1, dinv1 = _tc_prep(h1, degp1, alive1)
    acc1 = _sc_spmm(g1, src_p, dst_p, weff1, zero_nc)
    out1, score1 = _tc_outscore(acc1, dinv1, self1, p1[:, None])
    alive2 = _tc_sel1(score1, alive1)
    _, h2, _, _, _, r2 = _tc_next(out1, score1, alive2, batch_p, W2c,
                                  b2c[None, :])

    weff2, degp2 = _sc_deg(src_p, dst_p, weff1, alive2[:, 0])
    g2, self2, dinv2 = _tc_prep(h2, degp2, alive2)
    acc2 = _sc_spmm(g2, src_p, dst_p, weff2, zero_nc)
    out2, score2 = _tc_outscore(acc2, dinv2, self2, p2[:, None])
    alive3 = _tc_sel2(score2, alive2)
    _, _, _, _, r3 = _tc_last(out2, score2, alive3, batch_p)

    return _tc_mlp(r1, r2, r3, M1W, M1b[None, :], M2W, M2b[None, :],
                   M3W, M3b[None, :])


# speedup vs baseline: 13.4911x; 1.1171x over previous
import functools

import jax
import jax.numpy as jnp
from jax import lax
from jax.experimental import pallas as pl
from jax.experimental.pallas import tpu as pltpu
from jax.experimental.pallas import tpu_sc as plsc

N = 10000
NP = 10240
E = 320000
EPT = 10240
EP_TOTAL = EPT * 32
C = 128
B = 64
INT_MIN = -2147483648

_mesh = plsc.VectorSubcoreMesh(core_axis_name="c", subcore_axis_name="s")
_sc_params = pltpu.CompilerParams(needs_layout_passes=False)


@functools.partial(
    pl.kernel,
    out_type=(jax.ShapeDtypeStruct((EP_TOTAL,), jnp.float32),
              jax.ShapeDtypeStruct((32, NP), jnp.float32)),
    mesh=_mesh,
    scratch_types=[pltpu.VMEM((EPT,), jnp.int32),
                   pltpu.VMEM((EPT,), jnp.int32),
                   pltpu.VMEM((EPT,), jnp.float32),
                   pltpu.VMEM((NP,), jnp.float32),
                   pltpu.VMEM((NP,), jnp.float32),
                   pltpu.VMEM((EPT,), jnp.float32)],
    compiler_params=_sc_params,
)
def _sc_deg(src_h, dst_h, w_h, alive_h, weff_h, degp_h,
            src_v, dst_v, w_v, alive_v, deg_v, weff_v):
    wid = lax.axis_index("c") * 16 + lax.axis_index("s")
    base = wid * EPT
    pltpu.sync_copy(src_h.at[pl.ds(base, EPT)], src_v)
    pltpu.sync_copy(dst_h.at[pl.ds(base, EPT)], dst_v)
    pltpu.sync_copy(w_h.at[pl.ds(base, EPT)], w_v)
    pltpu.sync_copy(alive_h, alive_v)

    zeros16 = jnp.zeros((16,), jnp.float32)

    def zbody(i, carry):
        deg_v[pl.ds(i * 16, 16)] = zeros16
        return carry
    lax.fori_loop(0, NP // 16, zbody, 0)

    def ebody(i, carry):
        sl = pl.ds(i * 16, 16)
        s = src_v[sl]
        d = dst_v[sl]
        w = w_v[sl]
        a_s = plsc.load_gather(alive_v, [s])
        a_d = plsc.load_gather(alive_v, [d])
        we = w * a_s * a_d
        weff_v[sl] = we
        plsc.addupdate_scatter(deg_v, [d], we)
        return carry
    lax.fori_loop(0, EPT // 16, ebody, 0)

    pltpu.sync_copy(weff_v, weff_h.at[pl.ds(base, EPT)])
    pltpu.sync_copy(deg_v, degp_h.at[wid])


@functools.partial(
    pl.kernel,
    out_type=jax.ShapeDtypeStruct((2, NP, C), jnp.float32),
    mesh=_mesh,
    scratch_types=[pltpu.VMEM((EPT,), jnp.int32),
                   pltpu.VMEM((EPT,), jnp.int32),
                   pltpu.VMEM((EPT,), jnp.float32),
                   pltpu.VMEM((64, C), jnp.float32),
                   pltpu.VMEM((64, C), jnp.float32),
                   pltpu.VMEM((64,), jnp.int32),
                   pltpu.VMEM((64,), jnp.int32),
                   pltpu.VMEM_SHARED((NP, C), jnp.float32),
                   pltpu.SemaphoreType.DMA,
                   pltpu.SemaphoreType.DMA,
                   pltpu.SemaphoreType.DMA,
                   pltpu.SemaphoreType.DMA],
    compiler_params=_sc_params,
)
def _sc_spmm(g_h, src_h, dst_h, w_h, zero_h, acc_out,
             src_v, dst_v, w_v, rows0, rows1, st0, st1, acc_sh,
             gs0, gs1, ss0, ss1):
    cid = lax.axis_index("c")
    sid = lax.axis_index("s")
    wid = cid * 16 + sid
    base = wid * EPT
    rstripe = pl.ds(sid * (NP // 16), NP // 16)
    pltpu.sync_copy(zero_h.at[rstripe], acc_sh.at[rstripe])
    pltpu.sync_copy(src_h.at[pl.ds(base, EPT)], src_v)
    pltpu.sync_copy(dst_h.at[pl.ds(base, EPT)], dst_v)
    pltpu.sync_copy(w_h.at[pl.ds(base, EPT)], w_v)
    plsc.subcore_barrier()

    bufs = (rows0, rows1)
    stages = (st0, st1)
    gsems = (gs0, gs1)
    ssems = (ss0, ss1)
    KE = 64
    NCH = EPT // KE

    def gather(j, b, issue):
        src = g_h.at[src_v.at[pl.ds(j * KE, KE)]]
        if issue:
            pltpu.async_copy(src, bufs[b], gsems[b])
        else:
            pltpu.make_async_copy(src, bufs[b], gsems[b]).wait()

    def scatter(b, issue):
        dst = acc_sh.at[stages[b]]
        if issue:
            pltpu.async_copy(bufs[b], dst, ssems[b], add=True)
        else:
            pltpu.make_async_copy(bufs[b], dst, ssems[b]).wait()

    gather(0, 0, True)

    def jbody(j, carry):
        eb = j * KE
        for b in range(2):
            @pl.when(j % 2 == b)
            def _():
                gather(j, b, False)
                for b2 in range(KE // 16):
                    stages[b][pl.ds(b2 * 16, 16)] = dst_v[pl.ds(eb + b2 * 16, 16)]
                @pl.when(j + 1 < NCH)
                def _():
                    @pl.when(j >= 1)
                    def _():
                        scatter(1 - b, False)
                    gather(j + 1, 1 - b, True)

                @plsc.parallel_loop(0, KE, unroll=4)
                def _(e):
                    ws = plsc.load_gather(
                        w_v, [jnp.full((16,), eb + e, jnp.int32)])
                    for b3 in range(8):
                        cs = pl.ds(b3 * 16, 16)
                        bufs[b][e, cs] = bufs[b][e, cs] * ws
                scatter(b, True)
        return carry
    lax.fori_loop(0, NCH, jbody, 0)
    scatter((NCH - 1) % 2, False)
    scatter(NCH % 2, False)

    plsc.subcore_barrier()
    pltpu.sync_copy(acc_sh.at[rstripe], acc_out.at[cid, rstripe])


BR = 1280
GR = NP // BR
RB2 = 256
GR2 = NP // RB2
_NEG = -3.0e38


def _rows(width):
    return pl.BlockSpec((BR, width), lambda i: (i, 0))


def _full2(a, b):
    return pl.BlockSpec((a, b), lambda i: (0, 0))


def _prep_body(matmul, refs):
    if matmul:
        h_ref, w_ref, b_ref, degp_ref, alive_ref, g_ref, self_ref, dinv_ref = refs
        h = jnp.dot(h_ref[...], w_ref[...],
                    preferred_element_type=jnp.float32) + b_ref[...]
    else:
        h_ref, degp_ref, alive_ref, g_ref, self_ref, dinv_ref = refs
        h = h_ref[...]
    alive = alive_ref[...]
    deg = jnp.sum(degp_ref[...], axis=0)[:, None] + alive
    dinv = lax.rsqrt(jnp.maximum(deg, 1e-12))
    g_ref[...] = h * dinv
    self_ref[...] = h * (alive * dinv * dinv)
    dinv_ref[...] = dinv


_prep_out = [jax.ShapeDtypeStruct((NP, C), jnp.float32),
             jax.ShapeDtypeStruct((NP, C), jnp.float32),
             jax.ShapeDtypeStruct((NP, 1), jnp.float32)]
_prep_out_specs = [_rows(C), _rows(C), _rows(1)]

_tc_prep_mm = pl.pallas_call(
    lambda *refs: _prep_body(True, refs),
    grid=(GR,),
    in_specs=[_rows(C), _full2(C, C), _full2(1, C),
              pl.BlockSpec((32, BR), lambda i: (0, i)), _rows(1)],
    out_specs=_prep_out_specs,
    out_shape=_prep_out)

_tc_prep = pl.pallas_call(
    lambda *refs: _prep_body(False, refs),
    grid=(GR,),
    in_specs=[_rows(C), pl.BlockSpec((32, BR), lambda i: (0, i)), _rows(1)],
    out_specs=_prep_out_specs,
    out_shape=_prep_out)


def _outscore_body(acc_ref, dinv_ref, self_ref, p_ref, out_ref, score_ref):
    accp = acc_ref[...]
    out = jnp.maximum(dinv_ref[...] * (accp[0] + accp[1]) + self_ref[...], 0.0)
    out_ref[...] = out
    p = p_ref[...]
    pn = jnp.sqrt(jnp.sum(p * p))
    score_ref[...] = jnp.dot(out, p, preferred_element_type=jnp.float32) / (pn + 1e-12)


_tc_outscore = pl.pallas_call(
    _outscore_body,
    grid=(GR,),
    in_specs=[pl.BlockSpec((2, BR, C), lambda i: (0, i, 0)),
              _rows(1), _rows(C), _full2(C, 1)],
    out_specs=[_rows(C), _rows(1)],
    out_shape=[jax.ShapeDtypeStruct((NP, C), jnp.float32),
               jax.ShapeDtypeStruct((NP, 1), jnp.float32)])


def _sel_body(k, score_ref, alive_ref, sel_ref):
    score = score_ref[...]
    bits = lax.bitcast_convert_type(score, jnp.int32)
    key = jnp.where(bits >= 0, bits, bits ^ jnp.int32(0x7FFFFFFF))
    key = jnp.where(alive_ref[...] > 0, key, jnp.int32(INT_MIN))

    def bis(_, lohi):
        lo, hi = lohi
        mid = (lo & hi) + ((lo ^ hi) >> 1)
        cnt = jnp.sum((key >= mid).astype(jnp.int32))
        big = cnt >= k
        return jnp.where(big, mid, lo), jnp.where(big, hi, mid)
    lo, _ = lax.fori_loop(0, 32, bis,
                          (jnp.int32(INT_MIN), jnp.int32(2147483647)))
    tstar = lo
    c1 = jnp.sum((key > tstar).astype(jnp.int32))
    m = k - c1
    ties = key == tstar
    idx = lax.broadcasted_iota(jnp.int32, key.shape, 0)

    def bis2(_, lohi):
        lo2, hi2 = lohi
        mid = (lo2 + hi2) // 2
        cnt = jnp.sum((ties & (idx < mid)).astype(jnp.int32))
        ge = cnt >= m
        return jnp.where(ge, lo2, mid), jnp.where(ge, mid, hi2)
    _, u = lax.fori_loop(0, 15, bis2, (jnp.int32(0), jnp.int32(NP)))
    sel = (key > tstar) | (ties & (idx < u) & (m > 0))
    sel_ref[...] = sel.astype(jnp.float32)


def _mk_sel(k):
    return pl.pallas_call(
        functools.partial(_sel_body, k),
        out_shape=jax.ShapeDtypeStruct((NP, 1), jnp.float32))


_tc_sel0 = _mk_sel(5000)
_tc_sel1 = _mk_sel(2500)
_tc_sel2 = _mk_sel(1250)


def _gather_readout(has_next, refs):
    if has_next:
        (out_ref, score_ref, sel_ref, batch_ref, wn_ref, bn_ref,
         hnext_ref, hn_ref, msum_ref, cnt_ref, smax_ref, r_ref) = refs
    else:
        (out_ref, score_ref, sel_ref, batch_ref,
         hnext_ref, msum_ref, cnt_ref, smax_ref, r_ref) = refs
    i = pl.program_id(0)
    sel = sel_ref[...]
    hnext = out_ref[...] * (jnp.tanh(score_ref[...]) * sel)
    hnext_ref[...] = hnext
    if has_next:
        hn_ref[...] = jnp.dot(hnext, wn_ref[...],
                              preferred_element_type=jnp.float32) + bn_ref[...]

    @pl.when(i == 0)
    def _():
        msum_ref[...] = jnp.zeros_like(msum_ref)
        cnt_ref[...] = jnp.zeros_like(cnt_ref)
        smax_ref[...] = jnp.full_like(smax_ref, _NEG)

    bat = batch_ref[...]
    oh = ((jnp.reshape(bat, (1, RB2))
           == lax.broadcasted_iota(jnp.int32, (B, RB2), 0)).astype(jnp.float32)
          * jnp.reshape(sel, (1, RB2)))
    msum_ref[...] += jnp.dot(oh, hnext, preferred_element_type=jnp.float32)
    cnt_ref[...] += jnp.sum(oh, axis=1)[:, None]

    hm = jnp.where(sel > 0, hnext, _NEG)

    def mb(bb, carry):
        mrow = jnp.where(bat == bb, 0.0, _NEG)
        val = jnp.max(hm + mrow, axis=0)[None, :]
        cur = smax_ref[pl.ds(bb, 1), :]
        smax_ref[pl.ds(bb, 1), :] = jnp.maximum(cur, val)
        return carry
    lax.fori_loop(0, B, mb, 0, unroll=False)

    @pl.when(i == GR2 - 1)
    def _():
        cnt = cnt_ref[...]
        mean = msum_ref[...] / jnp.maximum(cnt, 1.0)
        smax = jnp.where(cnt > 0, smax_ref[...], 0.0)
        r_ref[...] = jnp.concatenate([mean, smax], axis=1)


def _rows2(width):
    return pl.BlockSpec((RB2, width), lambda i: (i, 0))


_gr_common_outs = [jax.ShapeDtypeStruct((B, C), jnp.float32),
                   jax.ShapeDtypeStruct((B, 1), jnp.float32),
                   jax.ShapeDtypeStruct((B, C), jnp.float32),
                   jax.ShapeDtypeStruct((B, 2 * C), jnp.float32)]
_gr_common_out_specs = [_full2(B, C), _full2(B, 1), _full2(B, C),
                        _full2(B, 2 * C)]

_tc_next = pl.pallas_call(
    lambda *refs: _gather_readout(True, refs),
    grid=(GR2,),
    in_specs=[_rows2(C), _rows2(1), _rows2(1), _rows2(1),
              _full2(C, C), _full2(1, C)],
    out_specs=[_rows2(C), _rows2(C)] + _gr_common_out_specs,
    out_shape=[jax.ShapeDtypeStruct((NP, C), jnp.float32),
               jax.ShapeDtypeStruct((NP, C), jnp.float32)] + _gr_common_outs)

_tc_last = pl.pallas_call(
    lambda *refs: _gather_readout(False, refs),
    grid=(GR2,),
    in_specs=[_rows2(C), _rows2(1), _rows2(1), _rows2(1)],
    out_specs=[_rows2(C)] + _gr_common_out_specs,
    out_shape=[jax.ShapeDtypeStruct((NP, C), jnp.float32)] + _gr_common_outs)


def _mlp_body(r0_ref, r1_ref, r2_ref, m1w_ref, m1b_ref, m2w_ref, m2b_ref,
              m3w_ref, m3b_ref, z_ref):
    r = r0_ref[...] + r1_ref[...] + r2_ref[...]
    z = jnp.maximum(jnp.dot(r, m1w_ref[...],
                            preferred_element_type=jnp.float32) + m1b_ref[...], 0.0)
    z = jnp.maximum(jnp.dot(z, m2w_ref[...],
                            preferred_element_type=jnp.float32) + m2b_ref[...], 0.0)
    z_ref[...] = jnp.dot(z, m3w_ref[...],
                         preferred_element_type=jnp.float32) + m3b_ref[...]


_tc_mlp = pl.pallas_call(
    _mlp_body, out_shape=jax.ShapeDtypeStruct((B, 1), jnp.float32))


def kernel(x, edge_index, edge_weight, batch, W0, b0, p0, W1c, b1c, p1,
           W2c, b2c, p2, M1W, M1b, M2W, M2b, M3W, M3b):
    f32 = jnp.float32
    src = edge_index[0]
    dst = edge_index[1]
    epad = EP_TOTAL - E
    src_p = jnp.concatenate([src, jnp.zeros((epad,), jnp.int32)])
    dst_p = jnp.concatenate([dst, jnp.zeros((epad,), jnp.int32)])
    w_p = jnp.concatenate([edge_weight.astype(f32), jnp.zeros((epad,), f32)])
    x_p = jnp.concatenate([x.astype(f32), jnp.zeros((NP - N, C), f32)], axis=0)
    batch_p = jnp.concatenate(
        [batch, jnp.full((NP - N,), B - 1, jnp.int32)])[:, None]
    alive0 = jnp.concatenate([jnp.ones((N, 1), f32), jnp.zeros((NP - N, 1), f32)])
    zero_nc = jnp.zeros((NP, C), f32)

    weff0, degp0 = _sc_deg(src_p, dst_p, w_p, alive0[:, 0])
    g0, self0, dinv0 = _tc_prep_mm(x_p, W0, b0[None, :], degp0, alive0)
    acc0 = _sc_spmm(g0, src_p, dst_p, weff0, zero_nc)
    out0, score0 = _tc_outscore(acc0, dinv0, self0, p0[:, None])
    alive1 = _tc_sel0(score0, alive0)
    _, h1, _, _, _, r1 = _tc_next(out0, score0, alive1, batch_p, W1c,
                                  b1c[None, :])

    weff1, degp1 = _sc_deg(src_p, dst_p, weff0, alive1[:, 0])
    g1, self1, dinv1 = _tc_prep(h1, degp1, alive1)
    acc1 = _sc_spmm(g1, src_p, dst_p, weff1, zero_nc)
    out1, score1 = _tc_outscore(acc1, dinv1, self1, p1[:, None])
    alive2 = _tc_sel1(score1, alive1)
    _, h2, _, _, _, r2 = _tc_next(out1, score1, alive2, batch_p, W2c,
                                  b2c[None, :])

    weff2, degp2 = _sc_deg(src_p, dst_p, weff1, alive2[:, 0])
    g2, self2, dinv2 = _tc_prep(h2, degp2, alive2)
    acc2 = _sc_spmm(g2, src_p, dst_p, weff2, zero_nc)
    out2, score2 = _tc_outscore(acc2, dinv2, self2, p2[:, None])
    alive3 = _tc_sel2(score2, alive2)
    _, _, _, _, r3 = _tc_last(out2, score2, alive3, batch_p)

    return _tc_mlp(r1, r2, r3, M1W, M1b[None, :], M2W, M2b[None, :],
                   M3W, M3b[None, :])


# speedup vs baseline: 14.3587x; 1.0643x over previous
import functools

import jax
import jax.numpy as jnp
from jax import lax
from jax.experimental import pallas as pl
from jax.experimental.pallas import tpu as pltpu
from jax.experimental.pallas import tpu_sc as plsc

N = 10000
NP = 10240
E = 320000
EPT = 10240
EP_TOTAL = EPT * 32
C = 128
B = 64
INT_MIN = -2147483648

_mesh = plsc.VectorSubcoreMesh(core_axis_name="c", subcore_axis_name="s")
_sc_params = pltpu.CompilerParams(needs_layout_passes=False)


@functools.partial(
    pl.kernel,
    out_type=(jax.ShapeDtypeStruct((EP_TOTAL,), jnp.float32),
              jax.ShapeDtypeStruct((32, NP), jnp.float32)),
    mesh=_mesh,
    scratch_types=[pltpu.VMEM((EPT,), jnp.int32),
                   pltpu.VMEM((EPT,), jnp.int32),
                   pltpu.VMEM((EPT,), jnp.float32),
                   pltpu.VMEM((NP,), jnp.float32),
                   pltpu.VMEM((NP,), jnp.float32),
                   pltpu.VMEM((EPT,), jnp.float32)],
    compiler_params=_sc_params,
)
def _sc_deg(src_h, dst_h, w_h, alive_h, weff_h, degp_h,
            src_v, dst_v, w_v, alive_v, deg_v, weff_v):
    wid = lax.axis_index("c") * 16 + lax.axis_index("s")
    base = wid * EPT
    pltpu.sync_copy(src_h.at[pl.ds(base, EPT)], src_v)
    pltpu.sync_copy(dst_h.at[pl.ds(base, EPT)], dst_v)
    pltpu.sync_copy(w_h.at[pl.ds(base, EPT)], w_v)
    pltpu.sync_copy(alive_h, alive_v)

    zeros16 = jnp.zeros((16,), jnp.float32)

    def zbody(i, carry):
        deg_v[pl.ds(i * 16, 16)] = zeros16
        return carry
    lax.fori_loop(0, NP // 16, zbody, 0)

    def ebody(i, carry):
        sl = pl.ds(i * 16, 16)
        s = src_v[sl]
        d = dst_v[sl]
        w = w_v[sl]
        a_s = plsc.load_gather(alive_v, [s])
        a_d = plsc.load_gather(alive_v, [d])
        we = w * a_s * a_d
        weff_v[sl] = we
        plsc.addupdate_scatter(deg_v, [d], we)
        return carry
    lax.fori_loop(0, EPT // 16, ebody, 0)

    pltpu.sync_copy(weff_v, weff_h.at[pl.ds(base, EPT)])
    pltpu.sync_copy(deg_v, degp_h.at[wid])


@functools.partial(
    pl.kernel,
    out_type=jax.ShapeDtypeStruct((2, NP, C), jnp.float32),
    mesh=_mesh,
    scratch_types=[pltpu.VMEM((EPT,), jnp.int32),
                   pltpu.VMEM((EPT,), jnp.int32),
                   pltpu.VMEM((EPT,), jnp.float32),
                   pltpu.VMEM((32, C), jnp.float32),
                   pltpu.VMEM((32, C), jnp.float32),
                   pltpu.VMEM((32, C), jnp.float32),
                   pltpu.VMEM((32, C), jnp.float32),
                   pltpu.VMEM((32,), jnp.int32),
                   pltpu.VMEM((32,), jnp.int32),
                   pltpu.VMEM((32,), jnp.int32),
                   pltpu.VMEM((32,), jnp.int32),
                   pltpu.VMEM_SHARED((NP, C), jnp.float32),
                   pltpu.SemaphoreType.DMA,
                   pltpu.SemaphoreType.DMA,
                   pltpu.SemaphoreType.DMA,
                   pltpu.SemaphoreType.DMA,
                   pltpu.SemaphoreType.DMA,
                   pltpu.SemaphoreType.DMA,
                   pltpu.SemaphoreType.DMA,
                   pltpu.SemaphoreType.DMA],
    compiler_params=_sc_params,
)
def _sc_spmm(g_h, src_h, dst_h, w_h, zero_h, acc_out,
             src_v, dst_v, w_v, rows0, rows1, rows2, rows3,
             st0, st1, st2, st3, acc_sh,
             gs0, gs1, gs2, gs3, ss0, ss1, ss2, ss3):
    cid = lax.axis_index("c")
    sid = lax.axis_index("s")
    wid = cid * 16 + sid
    base = wid * EPT
    rstripe = pl.ds(sid * (NP // 16), NP // 16)
    pltpu.sync_copy(zero_h.at[rstripe], acc_sh.at[rstripe])
    pltpu.sync_copy(src_h.at[pl.ds(base, EPT)], src_v)
    pltpu.sync_copy(dst_h.at[pl.ds(base, EPT)], dst_v)
    pltpu.sync_copy(w_h.at[pl.ds(base, EPT)], w_v)
    plsc.subcore_barrier()

    bufs = (rows0, rows1, rows2, rows3)
    stages = (st0, st1, st2, st3)
    gsems = (gs0, gs1, gs2, gs3)
    ssems = (ss0, ss1, ss2, ss3)
    KE = 32
    NB = 4
    PD = 2
    NCH = EPT // KE

    def gather(j, b, issue):
        src = g_h.at[src_v.at[pl.ds(j * KE, KE)]]
        if issue:
            pltpu.async_copy(src, bufs[b], gsems[b])
        else:
            pltpu.make_async_copy(src, bufs[b], gsems[b]).wait()

    def scatter(b, issue):
        dst = acc_sh.at[stages[b]]
        if issue:
            pltpu.async_copy(bufs[b], dst, ssems[b], add=True)
        else:
            pltpu.make_async_copy(bufs[b], dst, ssems[b]).wait()

    for b in range(PD):
        gather(b, b, True)

    def jbody(j, carry):
        eb = j * KE
        for b in range(NB):
            @pl.when(jnp.logical_and((j + PD) % NB == b, j + PD < NCH))
            def _():
                @pl.when(j + PD >= NB)
                def _():
                    scatter(b, False)
                gather(j + PD, b, True)
        for b in range(NB):
            @pl.when(j % NB == b)
            def _():
                gather(j, b, False)
                for b2 in range(KE // 16):
                    stages[b][pl.ds(b2 * 16, 16)] = dst_v[pl.ds(eb + b2 * 16, 16)]

                @plsc.parallel_loop(0, KE, unroll=4)
                def _(e):
                    ws = plsc.load_gather(
                        w_v, [jnp.full((16,), eb + e, jnp.int32)])
                    for b3 in range(8):
                        cs = pl.ds(b3 * 16, 16)
                        bufs[b][e, cs] = bufs[b][e, cs] * ws
                scatter(b, True)
        return carry
    lax.fori_loop(0, NCH, jbody, 0)
    for b in range(NB):
        scatter(b, False)

    plsc.subcore_barrier()
    pltpu.sync_copy(acc_sh.at[rstripe], acc_out.at[cid, rstripe])


BR = 1280
GR = NP // BR
RB2 = 256
GR2 = NP // RB2
_NEG = -3.0e38


def _rows(width):
    return pl.BlockSpec((BR, width), lambda i: (i, 0))


def _full2(a, b):
    return pl.BlockSpec((a, b), lambda i: (0, 0))


def _prep_body(matmul, refs):
    if matmul:
        h_ref, w_ref, b_ref, degp_ref, alive_ref, g_ref, self_ref, dinv_ref = refs
        h = jnp.dot(h_ref[...], w_ref[...],
                    preferred_element_type=jnp.float32) + b_ref[...]
    else:
        h_ref, degp_ref, alive_ref, g_ref, self_ref, dinv_ref = refs
        h = h_ref[...]
    alive = alive_ref[...]
    deg = jnp.sum(degp_ref[...], axis=0)[:, None] + alive
    dinv = lax.rsqrt(jnp.maximum(deg, 1e-12))
    g_ref[...] = h * dinv
    self_ref[...] = h * (alive * dinv * dinv)
    dinv_ref[...] = dinv


_prep_out = [jax.ShapeDtypeStruct((NP, C), jnp.float32),
             jax.ShapeDtypeStruct((NP, C), jnp.float32),
             jax.ShapeDtypeStruct((NP, 1), jnp.float32)]
_prep_out_specs = [_rows(C), _rows(C), _rows(1)]

_tc_prep_mm = pl.pallas_call(
    lambda *refs: _prep_body(True, refs),
    grid=(GR,),
    in_specs=[_rows(C), _full2(C, C), _full2(1, C),
              pl.BlockSpec((32, BR), lambda i: (0, i)), _rows(1)],
    out_specs=_prep_out_specs,
    out_shape=_prep_out)

_tc_prep = pl.pallas_call(
    lambda *refs: _prep_body(False, refs),
    grid=(GR,),
    in_specs=[_rows(C), pl.BlockSpec((32, BR), lambda i: (0, i)), _rows(1)],
    out_specs=_prep_out_specs,
    out_shape=_prep_out)


def _outscore_body(acc_ref, dinv_ref, self_ref, p_ref, out_ref, score_ref):
    accp = acc_ref[...]
    out = jnp.maximum(dinv_ref[...] * (accp[0] + accp[1]) + self_ref[...], 0.0)
    out_ref[...] = out
    p = p_ref[...]
    pn = jnp.sqrt(jnp.sum(p * p))
    score_ref[...] = jnp.dot(out, p, preferred_element_type=jnp.float32) / (pn + 1e-12)


_tc_outscore = pl.pallas_call(
    _outscore_body,
    grid=(GR,),
    in_specs=[pl.BlockSpec((2, BR, C), lambda i: (0, i, 0)),
              _rows(1), _rows(C), _full2(C, 1)],
    out_specs=[_rows(C), _rows(1)],
    out_shape=[jax.ShapeDtypeStruct((NP, C), jnp.float32),
               jax.ShapeDtypeStruct((NP, 1), jnp.float32)])


def _sel_body(k, score_ref, alive_ref, sel_ref):
    score = score_ref[...]
    bits = lax.bitcast_convert_type(score, jnp.int32)
    key = jnp.where(bits >= 0, bits, bits ^ jnp.int32(0x7FFFFFFF))
    key = jnp.where(alive_ref[...] > 0, key, jnp.int32(INT_MIN))

    def bis(_, lohi):
        lo, hi = lohi
        mid = (lo & hi) + ((lo ^ hi) >> 1)
        cnt = jnp.sum((key >= mid).astype(jnp.int32))
        big = cnt >= k
        return jnp.where(big, mid, lo), jnp.where(big, hi, mid)
    lo, _ = lax.fori_loop(0, 32, bis,
                          (jnp.int32(INT_MIN), jnp.int32(2147483647)))
    tstar = lo
    c1 = jnp.sum((key > tstar).astype(jnp.int32))
    m = k - c1
    ties = key == tstar
    idx = lax.broadcasted_iota(jnp.int32, key.shape, 0)

    def bis2(_, lohi):
        lo2, hi2 = lohi
        mid = (lo2 + hi2) // 2
        cnt = jnp.sum((ties & (idx < mid)).astype(jnp.int32))
        ge = cnt >= m
        return jnp.where(ge, lo2, mid), jnp.where(ge, mid, hi2)
    _, u = lax.fori_loop(0, 15, bis2, (jnp.int32(0), jnp.int32(NP)))
    sel = (key > tstar) | (ties & (idx < u) & (m > 0))
    sel_ref[...] = sel.astype(jnp.float32)


def _mk_sel(k):
    return pl.pallas_call(
        functools.partial(_sel_body, k),
        out_shape=jax.ShapeDtypeStruct((NP, 1), jnp.float32))


_tc_sel0 = _mk_sel(5000)
_tc_sel1 = _mk_sel(2500)
_tc_sel2 = _mk_sel(1250)


def _gather_readout(has_next, refs):
    if has_next:
        (out_ref, score_ref, sel_ref, batch_ref, wn_ref, bn_ref,
         hnext_ref, hn_ref, msum_ref, cnt_ref, smax_ref, r_ref) = refs
    else:
        (out_ref, score_ref, sel_ref, batch_ref,
         hnext_ref, msum_ref, cnt_ref, smax_ref, r_ref) = refs
    i = pl.program_id(0)
    sel = sel_ref[...]
    hnext = out_ref[...] * (jnp.tanh(score_ref[...]) * sel)
    hnext_ref[...] = hnext
    if has_next:
        hn_ref[...] = jnp.dot(hnext, wn_ref[...],
                              preferred_element_type=jnp.float32) + bn_ref[...]

    @pl.when(i == 0)
    def _():
        msum_ref[...] = jnp.zeros_like(msum_ref)
        cnt_ref[...] = jnp.zeros_like(cnt_ref)
        smax_ref[...] = jnp.full_like(smax_ref, _NEG)

    bat = batch_ref[...]
    oh = ((jnp.reshape(bat, (1, RB2))
           == lax.broadcasted_iota(jnp.int32, (B, RB2), 0)).astype(jnp.float32)
          * jnp.reshape(sel, (1, RB2)))
    msum_ref[...] += jnp.dot(oh, hnext, preferred_element_type=jnp.float32)
    cnt_ref[...] += jnp.sum(oh, axis=1)[:, None]

    hm = jnp.where(sel > 0, hnext, _NEG)

    def mb(bb, carry):
        mrow = jnp.where(bat == bb, 0.0, _NEG)
        val = jnp.max(hm + mrow, axis=0)[None, :]
        cur = smax_ref[pl.ds(bb, 1), :]
        smax_ref[pl.ds(bb, 1), :] = jnp.maximum(cur, val)
        return carry
    lax.fori_loop(0, B, mb, 0, unroll=False)

    @pl.when(i == GR2 - 1)
    def _():
        cnt = cnt_ref[...]
        mean = msum_ref[...] / jnp.maximum(cnt, 1.0)
        smax = jnp.where(cnt > 0, smax_ref[...], 0.0)
        r_ref[...] = jnp.concatenate([mean, smax], axis=1)


def _rows2(width):
    return pl.BlockSpec((RB2, width), lambda i: (i, 0))


_gr_common_outs = [jax.ShapeDtypeStruct((B, C), jnp.float32),
                   jax.ShapeDtypeStruct((B, 1), jnp.float32),
                   jax.ShapeDtypeStruct((B, C), jnp.float32),
                   jax.ShapeDtypeStruct((B, 2 * C), jnp.float32)]
_gr_common_out_specs = [_full2(B, C), _full2(B, 1), _full2(B, C),
                        _full2(B, 2 * C)]

_tc_next = pl.pallas_call(
    lambda *refs: _gather_readout(True, refs),
    grid=(GR2,),
    in_specs=[_rows2(C), _rows2(1), _rows2(1), _rows2(1),
              _full2(C, C), _full2(1, C)],
    out_specs=[_rows2(C), _rows2(C)] + _gr_common_out_specs,
    out_shape=[jax.ShapeDtypeStruct((NP, C), jnp.float32),
               jax.ShapeDtypeStruct((NP, C), jnp.float32)] + _gr_common_outs)

_tc_last = pl.pallas_call(
    lambda *refs: _gather_readout(False, refs),
    grid=(GR2,),
    in_specs=[_rows2(C), _rows2(1), _rows2(1), _rows2(1)],
    out_specs=[_rows2(C)] + _gr_common_out_specs,
    out_shape=[jax.ShapeDtypeStruct((NP, C), jnp.float32)] + _gr_common_outs)


def _mlp_body(r0_ref, r1_ref, r2_ref, m1w_ref, m1b_ref, m2w_ref, m2b_ref,
              m3w_ref, m3b_ref, z_ref):
    r = r0_ref[...] + r1_ref[...] + r2_ref[...]
    z = jnp.maximum(jnp.dot(r, m1w_ref[...],
                            preferred_element_type=jnp.float32) + m1b_ref[...], 0.0)
    z = jnp.maximum(jnp.dot(z, m2w_ref[...],
                            preferred_element_type=jnp.float32) + m2b_ref[...], 0.0)
    z_ref[...] = jnp.dot(z, m3w_ref[...],
                         preferred_element_type=jnp.float32) + m3b_ref[...]


_tc_mlp = pl.pallas_call(
    _mlp_body, out_shape=jax.ShapeDtypeStruct((B, 1), jnp.float32))


def kernel(x, edge_index, edge_weight, batch, W0, b0, p0, W1c, b1c, p1,
           W2c, b2c, p2, M1W, M1b, M2W, M2b, M3W, M3b):
    f32 = jnp.float32
    src = edge_index[0]
    dst = edge_index[1]
    epad = EP_TOTAL - E
    src_p = jnp.concatenate([src, jnp.zeros((epad,), jnp.int32)])
    dst_p = jnp.concatenate([dst, jnp.zeros((epad,), jnp.int32)])
    w_p = jnp.concatenate([edge_weight.astype(f32), jnp.zeros((epad,), f32)])
    x_p = jnp.concatenate([x.astype(f32), jnp.zeros((NP - N, C), f32)], axis=0)
    batch_p = jnp.concatenate(
        [batch, jnp.full((NP - N,), B - 1, jnp.int32)])[:, None]
    alive0 = jnp.concatenate([jnp.ones((N, 1), f32), jnp.zeros((NP - N, 1), f32)])
    zero_nc = jnp.zeros((NP, C), f32)

    weff0, degp0 = _sc_deg(src_p, dst_p, w_p, alive0[:, 0])
    g0, self0, dinv0 = _tc_prep_mm(x_p, W0, b0[None, :], degp0, alive0)
    acc0 = _sc_spmm(g0, src_p, dst_p, weff0, zero_nc)
    out0, score0 = _tc_outscore(acc0, dinv0, self0, p0[:, None])
    alive1 = _tc_sel0(score0, alive0)
    _, h1, _, _, _, r1 = _tc_next(out0, score0, alive1, batch_p, W1c,
                                  b1c[None, :])

    weff1, degp1 = _sc_deg(src_p, dst_p, weff0, alive1[:, 0])
    g1, self1, dinv1 = _tc_prep(h1, degp1, alive1)
    acc1 = _sc_spmm(g1, src_p, dst_p, weff1, zero_nc)
    out1, score1 = _tc_outscore(acc1, dinv1, self1, p1[:, None])
    alive2 = _tc_sel1(score1, alive1)
    _, h2, _, _, _, r2 = _tc_next(out1, score1, alive2, batch_p, W2c,
                                  b2c[None, :])

    weff2, degp2 = _sc_deg(src_p, dst_p, weff1, alive2[:, 0])
    g2, self2, dinv2 = _tc_prep(h2, degp2, alive2)
    acc2 = _sc_spmm(g2, src_p, dst_p, weff2, zero_nc)
    out2, score2 = _tc_outscore(acc2, dinv2, self2, p2[:, None])
    alive3 = _tc_sel2(score2, alive2)
    _, _, _, _, r3 = _tc_last(out2, score2, alive3, batch_p)

    return _tc_mlp(r1, r2, r3, M1W, M1b[None, :], M2W, M2b[None, :],
                   M3W, M3b[None, :])


# speedup vs baseline: 14.3625x; 1.0003x over previous
import functools

import jax
import jax.numpy as jnp
from jax import lax
from jax.experimental import pallas as pl
from jax.experimental.pallas import tpu as pltpu
from jax.experimental.pallas import tpu_sc as plsc

N = 10000
NP = 10240
E = 320000
EPT = 10240
EP_TOTAL = EPT * 32
C = 128
B = 64
INT_MIN = -2147483648

_mesh = plsc.VectorSubcoreMesh(core_axis_name="c", subcore_axis_name="s")
_sc_params = pltpu.CompilerParams(needs_layout_passes=False)


@functools.partial(
    pl.kernel,
    out_type=(jax.ShapeDtypeStruct((EP_TOTAL,), jnp.float32),
              jax.ShapeDtypeStruct((32, NP), jnp.float32)),
    mesh=_mesh,
    scratch_types=[pltpu.VMEM((EPT,), jnp.int32),
                   pltpu.VMEM((EPT,), jnp.int32),
                   pltpu.VMEM((EPT,), jnp.float32),
                   pltpu.VMEM((NP,), jnp.float32),
                   pltpu.VMEM((NP,), jnp.float32),
                   pltpu.VMEM((EPT,), jnp.float32)],
    compiler_params=_sc_params,
)
def _sc_deg(src_h, dst_h, w_h, alive_h, weff_h, degp_h,
            src_v, dst_v, w_v, alive_v, deg_v, weff_v):
    wid = lax.axis_index("c") * 16 + lax.axis_index("s")
    base = wid * EPT
    pltpu.sync_copy(src_h.at[pl.ds(base, EPT)], src_v)
    pltpu.sync_copy(dst_h.at[pl.ds(base, EPT)], dst_v)
    pltpu.sync_copy(w_h.at[pl.ds(base, EPT)], w_v)
    pltpu.sync_copy(alive_h, alive_v)

    zeros16 = jnp.zeros((16,), jnp.float32)

    def zbody(i, carry):
        deg_v[pl.ds(i * 16, 16)] = zeros16
        return carry
    lax.fori_loop(0, NP // 16, zbody, 0)

    def ebody(i, carry):
        sl = pl.ds(i * 16, 16)
        s = src_v[sl]
        d = dst_v[sl]
        w = w_v[sl]
        a_s = plsc.load_gather(alive_v, [s])
        a_d = plsc.load_gather(alive_v, [d])
        we = w * a_s * a_d
        weff_v[sl] = we
        plsc.addupdate_scatter(deg_v, [d], we)
        return carry
    lax.fori_loop(0, EPT // 16, ebody, 0)

    pltpu.sync_copy(weff_v, weff_h.at[pl.ds(base, EPT)])
    pltpu.sync_copy(deg_v, degp_h.at[wid])


@functools.partial(
    pl.kernel,
    out_type=jax.ShapeDtypeStruct((2, NP, C), jnp.float32),
    mesh=_mesh,
    scratch_types=[pltpu.VMEM((EPT,), jnp.int32),
                   pltpu.VMEM((EPT,), jnp.int32),
                   pltpu.VMEM((EPT,), jnp.float32),
                   pltpu.VMEM((32, C), jnp.float32),
                   pltpu.VMEM((32, C), jnp.float32),
                   pltpu.VMEM((32, C), jnp.float32),
                   pltpu.VMEM((32, C), jnp.float32),
                   pltpu.VMEM((32,), jnp.int32),
                   pltpu.VMEM((32,), jnp.int32),
                   pltpu.VMEM((32,), jnp.int32),
                   pltpu.VMEM((32,), jnp.int32),
                   pltpu.VMEM_SHARED((NP, C), jnp.float32),
                   pltpu.SemaphoreType.DMA,
                   pltpu.SemaphoreType.DMA,
                   pltpu.SemaphoreType.DMA,
                   pltpu.SemaphoreType.DMA,
                   pltpu.SemaphoreType.DMA,
                   pltpu.SemaphoreType.DMA,
                   pltpu.SemaphoreType.DMA,
                   pltpu.SemaphoreType.DMA],
    compiler_params=_sc_params,
)
def _sc_spmm(g_h, src_h, dst_h, w_h, zero_h, acc_out,
             src_v, dst_v, w_v, rows0, rows1, rows2, rows3,
             st0, st1, st2, st3, acc_sh,
             gs0, gs1, gs2, gs3, ss0, ss1, ss2, ss3):
    cid = lax.axis_index("c")
    sid = lax.axis_index("s")
    wid = cid * 16 + sid
    base = wid * EPT
    rstripe = pl.ds(sid * (NP // 16), NP // 16)
    pltpu.sync_copy(zero_h.at[rstripe], acc_sh.at[rstripe])
    pltpu.sync_copy(src_h.at[pl.ds(base, EPT)], src_v)
    pltpu.sync_copy(dst_h.at[pl.ds(base, EPT)], dst_v)
    pltpu.sync_copy(w_h.at[pl.ds(base, EPT)], w_v)
    plsc.subcore_barrier()

    bufs = (rows0, rows1, rows2, rows3)
    stages = (st0, st1, st2, st3)
    gsems = (gs0, gs1, gs2, gs3)
    ssems = (ss0, ss1, ss2, ss3)
    KE = 32
    NB = 4
    PD = 2
    NCH = EPT // KE

    def gather(j, b, issue):
        src = g_h.at[src_v.at[pl.ds(j * KE, KE)]]
        if issue:
            pltpu.async_copy(src, bufs[b], gsems[b])
        else:
            pltpu.make_async_copy(src, bufs[b], gsems[b]).wait()

    def scatter(b, issue):
        dst = acc_sh.at[stages[b]]
        if issue:
            pltpu.async_copy(bufs[b], dst, ssems[b], add=True)
        else:
            pltpu.make_async_copy(bufs[b], dst, ssems[b]).wait()

    for b in range(PD):
        gather(b, b, True)

    def jbody(j, carry):
        eb = j * KE
        for b in range(NB):
            @pl.when(jnp.logical_and((j + PD) % NB == b, j + PD < NCH))
            def _():
                @pl.when(j + PD >= NB)
                def _():
                    scatter(b, False)
                gather(j + PD, b, True)
        for b in range(NB):
            @pl.when(j % NB == b)
            def _():
                gather(j, b, False)
                for b2 in range(KE // 16):
                    stages[b][pl.ds(b2 * 16, 16)] = dst_v[pl.ds(eb + b2 * 16, 16)]

                @plsc.parallel_loop(0, KE, unroll=4)
                def _(e):
                    ws = plsc.load_gather(
                        w_v, [jnp.full((16,), eb + e, jnp.int32)])
                    for b3 in range(8):
                        cs = pl.ds(b3 * 16, 16)
                        bufs[b][e, cs] = bufs[b][e, cs] * ws
                scatter(b, True)
        return carry
    lax.fori_loop(0, NCH, jbody, 0)
    for b in range(NB):
        scatter(b, False)

    plsc.subcore_barrier()
    pltpu.sync_copy(acc_sh.at[rstripe], acc_out.at[cid, rstripe])


BR = 1280
GR = NP // BR
RB2 = 256
GR2 = NP // RB2
_NEG = -3.0e38


def _rows(width):
    return pl.BlockSpec((BR, width), lambda i: (i, 0))


def _full2(a, b):
    return pl.BlockSpec((a, b), lambda i: (0, 0))


def _prep_body(matmul, refs):
    if matmul:
        h_ref, w_ref, b_ref, degp_ref, alive_ref, g_ref, self_ref, dinv_ref = refs
        h = jnp.dot(h_ref[...], w_ref[...],
                    preferred_element_type=jnp.float32) + b_ref[...]
    else:
        h_ref, degp_ref, alive_ref, g_ref, self_ref, dinv_ref = refs
        h = h_ref[...]
    alive = alive_ref[...]
    deg = jnp.sum(degp_ref[...], axis=0)[:, None] + alive
    dinv = lax.rsqrt(jnp.maximum(deg, 1e-12))
    g_ref[...] = h * dinv
    self_ref[...] = h * (alive * dinv * dinv)
    dinv_ref[...] = dinv


_prep_out = [jax.ShapeDtypeStruct((NP, C), jnp.float32),
             jax.ShapeDtypeStruct((NP, C), jnp.float32),
             jax.ShapeDtypeStruct((NP, 1), jnp.float32)]
_prep_out_specs = [_rows(C), _rows(C), _rows(1)]

_tc_prep_mm = pl.pallas_call(
    lambda *refs: _prep_body(True, refs),
    grid=(GR,),
    in_specs=[_rows(C), _full2(C, C), _full2(1, C),
              pl.BlockSpec((32, BR), lambda i: (0, i)), _rows(1)],
    out_specs=_prep_out_specs,
    out_shape=_prep_out)

_tc_prep = pl.pallas_call(
    lambda *refs: _prep_body(False, refs),
    grid=(GR,),
    in_specs=[_rows(C), pl.BlockSpec((32, BR), lambda i: (0, i)), _rows(1)],
    out_specs=_prep_out_specs,
    out_shape=_prep_out)


def _outscore_body(acc_ref, dinv_ref, self_ref, p_ref, out_ref, score_ref):
    accp = acc_ref[...]
    out = jnp.maximum(dinv_ref[...] * (accp[0] + accp[1]) + self_ref[...], 0.0)
    out_ref[...] = out
    p = p_ref[...]
    pn = jnp.sqrt(jnp.sum(p * p))
    score_ref[...] = jnp.dot(out, p, preferred_element_type=jnp.float32) / (pn + 1e-12)


_tc_outscore = pl.pallas_call(
    _outscore_body,
    grid=(GR,),
    in_specs=[pl.BlockSpec((2, BR, C), lambda i: (0, i, 0)),
              _rows(1), _rows(C), _full2(C, 1)],
    out_specs=[_rows(C), _rows(1)],
    out_shape=[jax.ShapeDtypeStruct((NP, C), jnp.float32),
               jax.ShapeDtypeStruct((NP, 1), jnp.float32)])


def _sel_body(k, score_ref, alive_ref, sel_ref):
    score = score_ref[...]
    bits = lax.bitcast_convert_type(score, jnp.int32)
    key = jnp.where(bits >= 0, bits, bits ^ jnp.int32(0x7FFFFFFF))
    key = jnp.where(alive_ref[...] > 0, key, jnp.int32(INT_MIN))

    def bis(_, lohi):
        lo, hi = lohi
        mid = (lo & hi) + ((lo ^ hi) >> 1)
        cnt = jnp.sum((key >= mid).astype(jnp.int32))
        big = cnt >= k
        return jnp.where(big, mid, lo), jnp.where(big, hi, mid)
    lo, _ = lax.fori_loop(0, 32, bis,
                          (jnp.int32(INT_MIN), jnp.int32(2147483647)))
    tstar = lo
    c1 = jnp.sum((key > tstar).astype(jnp.int32))
    m = k - c1
    ties = key == tstar
    idx = lax.broadcasted_iota(jnp.int32, key.shape, 0)

    def bis2(_, lohi):
        lo2, hi2 = lohi
        mid = (lo2 + hi2) // 2
        cnt = jnp.sum((ties & (idx < mid)).astype(jnp.int32))
        ge = cnt >= m
        return jnp.where(ge, lo2, mid), jnp.where(ge, mid, hi2)
    _, u = lax.fori_loop(0, 15, bis2, (jnp.int32(0), jnp.int32(NP)))
    sel = (key > tstar) | (ties & (idx < u) & (m > 0))
    sel_ref[...] = sel.astype(jnp.float32)


def _mk_sel(k):
    return pl.pallas_call(
        functools.partial(_sel_body, k),
        out_shape=jax.ShapeDtypeStruct((NP, 1), jnp.float32))


_tc_sel0 = _mk_sel(5000)
_tc_sel1 = _mk_sel(2500)
_tc_sel2 = _mk_sel(1250)


def _gather_readout(has_next, refs):
    if has_next:
        (out_ref, score_ref, sel_ref, batch_ref, wn_ref, bn_ref,
         hnext_ref, hn_ref, msum_ref, cnt_ref, smax_ref, r_ref) = refs
    else:
        (out_ref, score_ref, sel_ref, batch_ref, r1_ref, r2_ref,
         m1w_ref, m1b_ref, m2w_ref, m2b_ref, m3w_ref, m3b_ref,
         hnext_ref, msum_ref, cnt_ref, smax_ref, r_ref) = refs
    i = pl.program_id(0)
    sel = sel_ref[...]
    hnext = out_ref[...] * (jnp.tanh(score_ref[...]) * sel)
    hnext_ref[...] = hnext
    if has_next:
        hn_ref[...] = jnp.dot(hnext, wn_ref[...],
                              preferred_element_type=jnp.float32) + bn_ref[...]

    @pl.when(i == 0)
    def _():
        msum_ref[...] = jnp.zeros_like(msum_ref)
        cnt_ref[...] = jnp.zeros_like(cnt_ref)
        smax_ref[...] = jnp.full_like(smax_ref, _NEG)

    bat = batch_ref[...]
    oh = ((jnp.reshape(bat, (1, RB2))
           == lax.broadcasted_iota(jnp.int32, (B, RB2), 0)).astype(jnp.float32)
          * jnp.reshape(sel, (1, RB2)))
    msum_ref[...] += jnp.dot(oh, hnext, preferred_element_type=jnp.float32)
    cnt_ref[...] += jnp.sum(oh, axis=1)[:, None]

    hm = jnp.where(sel > 0, hnext, _NEG)

    def mb(bb, carry):
        mrow = jnp.where(bat == bb, 0.0, _NEG)
        val = jnp.max(hm + mrow, axis=0)[None, :]
        cur = smax_ref[pl.ds(bb, 1), :]
        smax_ref[pl.ds(bb, 1), :] = jnp.maximum(cur, val)
        return carry
    lax.fori_loop(0, B, mb, 0, unroll=False)

    @pl.when(i == GR2 - 1)
    def _():
        cnt = cnt_ref[...]
        mean = msum_ref[...] / jnp.maximum(cnt, 1.0)
        smax = jnp.where(cnt > 0, smax_ref[...], 0.0)
        r = jnp.concatenate([mean, smax], axis=1)
        if has_next:
            r_ref[...] = r
        else:
            r = r + r1_ref[...] + r2_ref[...]
            z = jnp.maximum(jnp.dot(r, m1w_ref[...],
                                    preferred_element_type=jnp.float32)
                            + m1b_ref[...], 0.0)
            z = jnp.maximum(jnp.dot(z, m2w_ref[...],
                                    preferred_element_type=jnp.float32)
                            + m2b_ref[...], 0.0)
            r_ref[...] = jnp.dot(z, m3w_ref[...],
                                 preferred_element_type=jnp.float32) + m3b_ref[...]


def _rows2(width):
    return pl.BlockSpec((RB2, width), lambda i: (i, 0))


_gr_common_outs = [jax.ShapeDtypeStruct((B, C), jnp.float32),
                   jax.ShapeDtypeStruct((B, 1), jnp.float32),
                   jax.ShapeDtypeStruct((B, C), jnp.float32),
                   jax.ShapeDtypeStruct((B, 2 * C), jnp.float32)]
_gr_common_out_specs = [_full2(B, C), _full2(B, 1), _full2(B, C),
                        _full2(B, 2 * C)]

_tc_next = pl.pallas_call(
    lambda *refs: _gather_readout(True, refs),
    grid=(GR2,),
    in_specs=[_rows2(C), _rows2(1), _rows2(1), _rows2(1),
              _full2(C, C), _full2(1, C)],
    out_specs=[_rows2(C), _rows2(C)] + _gr_common_out_specs,
    out_shape=[jax.ShapeDtypeStruct((NP, C), jnp.float32),
               jax.ShapeDtypeStruct((NP, C), jnp.float32)] + _gr_common_outs)

_tc_last = pl.pallas_call(
    lambda *refs: _gather_readout(False, refs),
    grid=(GR2,),
    in_specs=[_rows2(C), _rows2(1), _rows2(1), _rows2(1),
              _full2(B, 2 * C), _full2(B, 2 * C),
              _full2(2 * C, C), _full2(1, C), _full2(C, C // 2),
              _full2(1, C // 2), _full2(C // 2, 1), _full2(1, 1)],
    out_specs=[_rows2(C)] + _gr_common_out_specs[:-1] + [_full2(B, 1)],
    out_shape=[jax.ShapeDtypeStruct((NP, C), jnp.float32)]
    + _gr_common_outs[:-1] + [jax.ShapeDtypeStruct((B, 1), jnp.float32)])


def _mlp_body(r0_ref, r1_ref, r2_ref, m1w_ref, m1b_ref, m2w_ref, m2b_ref,
              m3w_ref, m3b_ref, z_ref):
    r = r0_ref[...] + r1_ref[...] + r2_ref[...]
    z = jnp.maximum(jnp.dot(r, m1w_ref[...],
                            preferred_element_type=jnp.float32) + m1b_ref[...], 0.0)
    z = jnp.maximum(jnp.dot(z, m2w_ref[...],
                            preferred_element_type=jnp.float32) + m2b_ref[...], 0.0)
    z_ref[...] = jnp.dot(z, m3w_ref[...],
                         preferred_element_type=jnp.float32) + m3b_ref[...]


_tc_mlp = pl.pallas_call(
    _mlp_body, out_shape=jax.ShapeDtypeStruct((B, 1), jnp.float32))


def kernel(x, edge_index, edge_weight, batch, W0, b0, p0, W1c, b1c, p1,
           W2c, b2c, p2, M1W, M1b, M2W, M2b, M3W, M3b):
    f32 = jnp.float32
    src = edge_index[0]
    dst = edge_index[1]
    epad = EP_TOTAL - E
    src_p = jnp.concatenate([src, jnp.zeros((epad,), jnp.int32)])
    dst_p = jnp.concatenate([dst, jnp.zeros((epad,), jnp.int32)])
    w_p = jnp.concatenate([edge_weight.astype(f32), jnp.zeros((epad,), f32)])
    x_p = jnp.concatenate([x.astype(f32), jnp.zeros((NP - N, C), f32)], axis=0)
    batch_p = jnp.concatenate(
        [batch, jnp.full((NP - N,), B - 1, jnp.int32)])[:, None]
    alive0 = jnp.concatenate([jnp.ones((N, 1), f32), jnp.zeros((NP - N, 1), f32)])
    zero_nc = jnp.zeros((NP, C), f32)

    weff0, degp0 = _sc_deg(src_p, dst_p, w_p, alive0[:, 0])
    g0, self0, dinv0 = _tc_prep_mm(x_p, W0, b0[None, :], degp0, alive0)
    acc0 = _sc_spmm(g0, src_p, dst_p, weff0, zero_nc)
    out0, score0 = _tc_outscore(acc0, dinv0, self0, p0[:, None])
    alive1 = _tc_sel0(score0, alive0)
    _, h1, _, _, _, r1 = _tc_next(out0, score0, alive1, batch_p, W1c,
                                  b1c[None, :])

    weff1, degp1 = _sc_deg(src_p, dst_p, weff0, alive1[:, 0])
    g1, self1, dinv1 = _tc_prep(h1, degp1, alive1)
    acc1 = _sc_spmm(g1, src_p, dst_p, weff1, zero_nc)
    out1, score1 = _tc_outscore(acc1, dinv1, self1, p1[:, None])
    alive2 = _tc_sel1(score1, alive1)
    _, h2, _, _, _, r2 = _tc_next(out1, score1, alive2, batch_p, W2c,
                                  b2c[None, :])

    weff2, degp2 = _sc_deg(src_p, dst_p, weff1, alive2[:, 0])
    g2, self2, dinv2 = _tc_prep(h2, degp2, alive2)
    acc2 = _sc_spmm(g2, src_p, dst_p, weff2, zero_nc)
    out2, score2 = _tc_outscore(acc2, dinv2, self2, p2[:, None])
    alive3 = _tc_sel2(score2, alive2)
    _, _, _, _, z = _tc_last(out2, score2, alive3, batch_p, r1, r2,
                             M1W, M1b[None, :], M2W, M2b[None, :],
                             M3W, M3b[None, :])
    return z
